# Initial kernel scaffold; baseline (speedup 1.0000x reference)
import jax
import jax.numpy as jnp
from jax.experimental import pallas as pl


def kernel(x, q, subj_idx, rel_idx, obj_idx, W1, b1, W2, b2, W3, b3, n_hop):
    raise NotImplementedError("write your pallas kernel here")


# speedup vs baseline: 22.7859x; 22.7859x over previous
import functools
import math

import jax
import jax.numpy as jnp
from jax import lax
from jax.experimental import pallas as pl
from jax.experimental.pallas import tpu as pltpu
from jax.experimental.pallas import tpu_sc as plsc

F32 = jnp.float32
I32 = jnp.int32

NC = 2
NS = 16
NW = NC * NS
CH = 512
G = 128
KSUB = CH // G
W16 = 16


def _r_body(q2_ref, w1, b1, w2, b2, w3, b3, r1, r2, r3):
    q2 = q2_ref[...]
    r1[...] = jnp.dot(w1[...], q2, preferred_element_type=F32) + b1[...]
    r2[...] = jnp.dot(w2[...], q2, preferred_element_type=F32) + b2[...]
    r3[...] = jnp.dot(w3[...], q2, preferred_element_type=F32) + b3[...]


def _compute_r(q, W1, b1, W2, b2, W3, b3):
    nr = W1.shape[0]
    qT = q.T
    q2 = jnp.concatenate([qT, qT], axis=1)
    out = jax.ShapeDtypeStruct((nr, W16), F32)
    return pl.pallas_call(_r_body, out_shape=[out, out, out])(
        q2, W1, b1.reshape(nr, 1), W2, b2.reshape(nr, 1), W3, b3.reshape(nr, 1))


def _add_body(a_ref, b_ref, o_ref):
    o_ref[...] = a_ref[...] + b_ref[...]


def _combine(p0, p1):
    xp, w = p0.shape
    a0 = p0.reshape(-1, 128)
    a1 = p1.reshape(-1, 128)
    out = pl.pallas_call(
        _add_body, out_shape=jax.ShapeDtypeStruct(a0.shape, F32))(a0, a1)
    return out.reshape(xp, w)


def _make_hop(XP, CPT):
    rows_pt = XP // NS
    mesh = plsc.VectorSubcoreMesh(core_axis_name="c", subcore_axis_name="s")
    part = jax.ShapeDtypeStruct((XP, W16), F32)

    @functools.partial(
        pl.kernel,
        out_type=[part, part],
        mesh=mesh,
        compiler_params=pltpu.CompilerParams(use_tc_tiling_on_sc=False),
        scratch_types=[
            pltpu.VMEM_SHARED((XP, W16), F32),
            pltpu.VMEM((KSUB, G), I32),
            pltpu.VMEM((KSUB, G), I32),
            pltpu.VMEM((KSUB, G), I32),
            pltpu.VMEM((CH, W16), F32),
            pltpu.VMEM((CH, W16), F32),
            pltpu.SemaphoreType.DMA,
        ],
    )
    def hop(x16, r16, subj, rel, obj, z, p0, p1, acc, si, ri, oi, xb, rb, semg):
        c = lax.axis_index("c")
        s = lax.axis_index("s")
        wid = c * NS + s

        pltpu.sync_copy(z, acc.at[pl.ds(s * rows_pt, rows_pt)])
        plsc.subcore_barrier()

        def chunk(h, carry):
            rowbase = wid * (CPT * KSUB) + h * KSUB
            pltpu.sync_copy(subj.at[pl.ds(rowbase, KSUB)], si)
            pltpu.sync_copy(rel.at[pl.ds(rowbase, KSUB)], ri)
            pltpu.sync_copy(obj.at[pl.ds(rowbase, KSUB)], oi)
            cps = []
            for j in range(KSUB):
                cps.append(pltpu.async_copy(
                    x16.at[si.at[j]], xb.at[pl.ds(j * G, G)], semg))
                cps.append(pltpu.async_copy(
                    r16.at[ri.at[j]], rb.at[pl.ds(j * G, G)], semg))
            for cp in cps:
                cp.wait()

            UNROLL = 8
            def mul_body(g, carry2):
                base = g * UNROLL
                for u in range(UNROLL):
                    xb[base + u, :] = xb[base + u, :] * rb[base + u, :]
                return carry2
            lax.fori_loop(0, CH // UNROLL, mul_body, 0)

            for j in range(KSUB):
                pltpu.sync_copy(xb.at[pl.ds(j * G, G)], acc.at[oi.at[j]],
                                add=True)
            return carry

        lax.fori_loop(0, CPT, chunk, 0)

        plsc.subcore_barrier()

        @pl.when(c == 0)
        def _():
            pltpu.sync_copy(acc.at[pl.ds(s * rows_pt, rows_pt)],
                            p0.at[pl.ds(s * rows_pt, rows_pt)])

        @pl.when(c == 1)
        def _():
            pltpu.sync_copy(acc.at[pl.ds(s * rows_pt, rows_pt)],
                            p1.at[pl.ds(s * rows_pt, rows_pt)])

    return hop


def kernel(x, q, subj_idx, rel_idx, obj_idx, W1, b1, W2, b2, W3, b3, n_hop):
    B, NE = x.shape
    NT = subj_idx.shape[0]
    CPT = math.ceil(NT / (NW * CH))
    NT_PAD = NW * CPT * CH
    XP = ((NE + 1 + 127) // 128) * 128
    pad = NT_PAD - NT

    subj = jnp.concatenate([subj_idx, jnp.zeros((pad,), I32)]).reshape(-1, G)
    rel = jnp.concatenate([rel_idx, jnp.zeros((pad,), I32)]).reshape(-1, G)
    obj = jnp.concatenate([obj_idx, jnp.full((pad,), NE, I32)]).reshape(-1, G)
    z = jnp.zeros((XP // NS, W16), F32)

    r1, r2, r3 = _compute_r(q, W1, b1, W2, b2, W3, b3)

    xT = x.T
    x16 = jnp.concatenate(
        [xT, xT], axis=1)
    x16 = jnp.concatenate(
        [x16, jnp.zeros((XP - NE, W16), F32)], axis=0)

    hop = _make_hop(XP, CPT)

    def do_hop(xt, rt):
        p0, p1 = hop(xt, rt, subj, rel, obj, z)
        return _combine(p0, p1)

    x16 = lax.cond(1 <= n_hop, lambda t: do_hop(t, r1), lambda t: t, x16)
    x16 = lax.cond(2 <= n_hop, lambda t: do_hop(t, r2), lambda t: t, x16)
    x16 = lax.cond(3 <= n_hop, lambda t: do_hop(t, r3), lambda t: t, x16)

    return x16[:NE, :B].T
